# baseline (speedup 1.0000x reference)
import functools

import jax
import jax.numpy as jnp
from jax import lax
from jax.experimental import pallas as pl
from jax.experimental.pallas import tpu as pltpu
from jax.experimental.pallas import tpu_sc as plsc

H = 768
E = 8
INTER = 2048
T = 2048
TP = 2 * T
TM = 256
NT = 23
NPAD = NT * TM
NW = 32
GPW = NPAD // NW
CPW = T // NW

_DEFAULT = jax.lax.Precision.DEFAULT


def _router_body(x_ref, wg_ref, idx_ref, w_ref):
    xf = x_ref[...]
    logits = jnp.dot(xf, wg_ref[...], preferred_element_type=jnp.float32)
    colid = jax.lax.broadcasted_iota(jnp.int32, logits.shape, 1)
    m1 = jnp.max(logits, axis=1, keepdims=True)
    idx1 = jnp.min(jnp.where(logits == m1, colid, E), axis=1, keepdims=True)
    l2 = jnp.where(colid == idx1, -jnp.inf, logits)
    m2 = jnp.max(l2, axis=1, keepdims=True)
    idx2 = jnp.min(jnp.where(l2 == m2, colid, E), axis=1, keepdims=True)
    t = jnp.exp(m2 - m1)
    w_top = 1.0 / (1.0 + t)
    w_sec = t / (1.0 + t)
    idx_ref[...] = jnp.concatenate([idx1, idx2], axis=1)
    w_ref[...] = jnp.concatenate([w_top, w_sec], axis=1)


def _router(x2d, Wg):
    return pl.pallas_call(
        _router_body,
        out_shape=[
            jax.ShapeDtypeStruct((T, 2), jnp.int32),
            jax.ShapeDtypeStruct((T, 2), jnp.float32),
        ],
    )(x2d, Wg)


_sc_mesh = plsc.VectorSubcoreMesh(core_axis_name="c", subcore_axis_name="s")

_GCH = ((0, 48), (48, 48), (96, 48), (144, 40))


@functools.partial(
    pl.kernel, mesh=_sc_mesh,
    out_type=jax.ShapeDtypeStruct((NPAD, H), jnp.float32),
    scratch_types=[
        pltpu.VMEM((48,), jnp.int32),
        pltpu.VMEM((48,), jnp.int32),
        pltpu.VMEM((48,), jnp.int32),
        pltpu.VMEM((40,), jnp.int32),
        pltpu.VMEM((48, H), jnp.float32),
        pltpu.VMEM((48, H), jnp.float32),
        pltpu.SemaphoreType.DMA,
        pltpu.SemaphoreType.DMA,
        pltpu.SemaphoreType.DMA,
        pltpu.SemaphoreType.DMA,
        pltpu.SemaphoreType.DMA,
    ],
)
def _sc_gather(x_hbm, idx_hbm, xs_hbm, i0, i1, i2, i3, buf0, buf1,
               isem, gs0, gs1, os0, os1):
    wid = lax.axis_index("s") * 2 + lax.axis_index("c")
    base = wid * GPW
    idxs = (i0, i1, i2, i3)
    bufs, gss, oss = (buf0, buf1), (gs0, gs1), (os0, os1)
    iw = []
    for c, (off, n) in enumerate(_GCH):
        iw.append(pltpu.async_copy(
            idx_hbm.at[pl.ds(base + off, n)], idxs[c], isem))
    for w in iw:
        w.wait()

    def gather(c):
        off, n = _GCH[c]
        b = bufs[c % 2]
        return pltpu.async_copy(
            x_hbm.at[idxs[c]], b.at[pl.ds(0, n)], gss[c % 2])

    def copyout(c):
        off, n = _GCH[c]
        b = bufs[c % 2]
        return pltpu.async_copy(
            b.at[pl.ds(0, n)], xs_hbm.at[pl.ds(base + off, n)], oss[c % 2])

    g0, g1 = gather(0), gather(1)
    g0.wait()
    o0 = copyout(0)
    g1.wait()
    o1 = copyout(1)
    o0.wait()
    g2 = gather(2)
    o1.wait()
    g3 = gather(3)
    g2.wait()
    o2 = copyout(2)
    g3.wait()
    o3 = copyout(3)
    o2.wait()
    o3.wait()


def _group_body(sched_ref, xs_ref, w_ref, w1_ref, w3_ref, w2_ref, ys_ref):
    xt = xs_ref[...]
    a = jnp.dot(xt, w1_ref[0], preferred_element_type=jnp.float32,
                precision=_DEFAULT)
    b = jnp.dot(xt, w3_ref[0], preferred_element_type=jnp.float32,
                precision=_DEFAULT)
    h = (a * jax.nn.sigmoid(a)) * b * w_ref[...]
    ys_ref[...] = jnp.dot(h, w2_ref[0], preferred_element_type=jnp.float32,
                          precision=_DEFAULT)


@jax.jit
def _grouped(sched, xs, w_pad, W1, W3, W2):
    grid_spec = pltpu.PrefetchScalarGridSpec(
        num_scalar_prefetch=1,
        grid=(NT,),
        in_specs=[
            pl.BlockSpec((TM, H), lambda s, sched: (s, 0)),
            pl.BlockSpec((TM, 1), lambda s, sched: (s, 0)),
            pl.BlockSpec((1, H, INTER), lambda s, sched: (sched[s], 0, 0)),
            pl.BlockSpec((1, H, INTER), lambda s, sched: (sched[s], 0, 0)),
            pl.BlockSpec((1, INTER, H), lambda s, sched: (sched[s], 0, 0)),
        ],
        out_specs=pl.BlockSpec((TM, H), lambda s, sched: (s, 0)),
    )
    return pl.pallas_call(
        _group_body,
        grid_spec=grid_spec,
        out_shape=jax.ShapeDtypeStruct((NPAD, H), jnp.float32),
        compiler_params=pltpu.CompilerParams(
            dimension_semantics=("arbitrary",),
        ),
    )(sched, xs, w_pad, W1, W3, W2)


_NCC = 4
_CT = CPW // _NCC


@functools.partial(
    pl.kernel, mesh=_sc_mesh,
    out_type=jax.ShapeDtypeStruct((T, H), jnp.float32),
    scratch_types=[
        pltpu.VMEM((2 * _CT,), jnp.int32),
        pltpu.VMEM((2 * _CT,), jnp.int32),
        pltpu.VMEM((2 * _CT, H), jnp.float32),
        pltpu.VMEM((2 * _CT, H), jnp.float32),
        pltpu.VMEM((_CT, H), jnp.float32),
        pltpu.VMEM((_CT, H), jnp.float32),
        pltpu.SemaphoreType.DMA,
        pltpu.SemaphoreType.DMA,
        pltpu.SemaphoreType.DMA,
        pltpu.SemaphoreType.DMA,
    ],
)
def _sc_combine(ys_hbm, pos_hbm, out_hbm, pidx0, pidx1, rows0, rows1,
                outb0, outb1, gs0, gs1, os0, os1):
    wid = lax.axis_index("s") * 2 + lax.axis_index("c")
    pidxs, rows, outbs = (pidx0, pidx1), (rows0, rows1), (outb0, outb1)
    gss, oss = (gs0, gs1), (os0, os1)

    def cgather(c):
        b = c % 2
        pbase = wid * 2 * CPW + c * 2 * _CT
        pltpu.sync_copy(pos_hbm.at[pl.ds(pbase, 2 * _CT)], pidxs[b])
        return pltpu.async_copy(ys_hbm.at[pidxs[b]], rows[b], gss[b])

    def compute(c):
        b = c % 2
        rv, ov = rows[b], outbs[b]

        def body(i, carry):
            for j in range(H // 16):
                s = 16 * j
                ov[i, pl.ds(s, 16)] = (
                    rv[2 * i, pl.ds(s, 16)] + rv[2 * i + 1, pl.ds(s, 16)])
            return carry

        lax.fori_loop(0, _CT, body, 0)
        return pltpu.async_copy(
            outbs[b], out_hbm.at[pl.ds(wid * CPW + c * _CT, _CT)], oss[b])

    g0, g1 = cgather(0), cgather(1)
    g0.wait()
    o0 = compute(0)
    g2 = cgather(2)
    g1.wait()
    o1 = compute(1)
    g3 = cgather(3)
    g2.wait()
    o0.wait()
    o2 = compute(2)
    g3.wait()
    o1.wait()
    o3 = compute(3)
    o2.wait()
    o3.wait()


@jax.jit
def _moe(x2d, Wg, W1, W3, W2):
    idx12, w12 = _router(x2d, Wg)
    ids = idx12.reshape(TP)
    pw = w12.reshape(TP)
    onehot = (ids[:, None] == jnp.arange(E, dtype=jnp.int32)[None, :]).astype(
        jnp.int32)
    inc = jnp.cumsum(onehot, axis=0)
    rank = jnp.take_along_axis(inc, ids[:, None], axis=1)[:, 0] - 1
    counts = inc[-1]
    tiles = (counts + TM - 1) // TM
    tile_cum = jnp.cumsum(tiles)
    row_start = (tile_cum - tiles) * TM
    ppos = (row_start[ids] + rank).astype(jnp.int32)
    sched = jnp.minimum(
        (jnp.arange(NT, dtype=jnp.int32)[:, None] >= tile_cum[None, :]).sum(
            axis=1), E - 1).astype(jnp.int32)
    tok = (jnp.arange(TP, dtype=jnp.int32) // 2).astype(jnp.int32)
    ids_pad = jnp.zeros((NPAD,), jnp.int32).at[ppos].set(
        tok, mode="drop", unique_indices=True)
    w_pad = jnp.zeros((NPAD,), jnp.float32).at[ppos].set(
        pw, mode="drop", unique_indices=True)

    xs = x2d[ids_pad]
    ys = _grouped(sched, xs, w_pad.reshape(NPAD, 1), W1, W3, W2)
    return _sc_combine(ys, ppos)


def kernel(x, Wg, W1, W3, W2):
    B, S, Hd = x.shape
    out = _moe(x.reshape(-1, Hd), Wg, W1, W3, W2)
    return out.reshape(B, S, Hd)


# speedup vs baseline: 1.2658x; 1.2658x over previous
import functools

import jax
import jax.numpy as jnp
from jax.experimental import pallas as pl
from jax.experimental.pallas import tpu as pltpu

H = 768
E = 8
INTER = 2048
TN = 1024
NI = INTER // TN


def _moe_body(x_ref, wg_ref, w1_ref, w3_ref, w2_ref, out_ref, rw_ref):
    e = pl.program_id(0)
    ni = pl.program_id(1)

    @pl.when((e == 0) & (ni == 0))
    def _init():
        xf = x_ref[...]
        logits = jnp.dot(xf, wg_ref[...], preferred_element_type=jnp.float32)
        colid = jax.lax.broadcasted_iota(jnp.int32, logits.shape, 1)
        m1 = jnp.max(logits, axis=1, keepdims=True)
        idx1 = jnp.min(jnp.where(logits == m1, colid, E), axis=1, keepdims=True)
        sel1 = colid == idx1
        l2 = jnp.where(sel1, -jnp.inf, logits)
        m2 = jnp.max(l2, axis=1, keepdims=True)
        idx2 = jnp.min(jnp.where(l2 == m2, colid, E), axis=1, keepdims=True)
        sel2 = colid == idx2
        t = jnp.exp(m2 - m1)
        w_top = 1.0 / (1.0 + t)
        w_sec = t / (1.0 + t)
        rw_ref[...] = jnp.where(sel1, w_top, jnp.where(sel2, w_sec, 0.0))
        out_ref[...] = jnp.zeros_like(out_ref)

    xb = x_ref[...]
    a = jnp.dot(xb, w1_ref[0], preferred_element_type=jnp.float32,
                precision=jax.lax.Precision.DEFAULT)
    b = jnp.dot(xb, w3_ref[0], preferred_element_type=jnp.float32,
                precision=jax.lax.Precision.DEFAULT)
    h = (a * jax.nn.sigmoid(a)) * b
    colid = jax.lax.broadcasted_iota(jnp.int32, rw_ref.shape, 1)
    w_col = jnp.sum(jnp.where(colid == e, rw_ref[...], 0.0), axis=1, keepdims=True)
    out_ref[...] += jnp.dot(
        h * w_col, w2_ref[0],
        preferred_element_type=jnp.float32,
        precision=jax.lax.Precision.DEFAULT)


@jax.jit
def _moe(xf, Wg, W1, W3, W2):
    T = xf.shape[0]
    return pl.pallas_call(
        _moe_body,
        grid=(E, NI),
        in_specs=[
            pl.BlockSpec((T, H), lambda e, ni: (0, 0)),
            pl.BlockSpec((H, E), lambda e, ni: (0, 0)),
            pl.BlockSpec((1, H, TN), lambda e, ni: (e, 0, ni)),
            pl.BlockSpec((1, H, TN), lambda e, ni: (e, 0, ni)),
            pl.BlockSpec((1, TN, H), lambda e, ni: (e, ni, 0)),
        ],
        out_specs=pl.BlockSpec((T, H), lambda e, ni: (0, 0)),
        out_shape=jax.ShapeDtypeStruct((T, H), jnp.float32),
        scratch_shapes=[pltpu.VMEM((T, E), jnp.float32)],
        compiler_params=pltpu.CompilerParams(
            dimension_semantics=("arbitrary", "arbitrary"),
        ),
    )(xf, Wg, W1, W3, W2)


def kernel(x, Wg, W1, W3, W2):
    B, S, Hd = x.shape
    xf = x.reshape(-1, Hd)
    out = _moe(xf, Wg, W1, W3, W2)
    return out.reshape(B, S, Hd)


# speedup vs baseline: 1.2712x; 1.0043x over previous
import functools

import jax
import jax.numpy as jnp
from jax.experimental import pallas as pl
from jax.experimental.pallas import tpu as pltpu

H = 768
E = 8
INTER = 2048
TN = 1024
NI = INTER // TN


def _moe_body(x_ref, wg_ref, w1_ref, w3_ref, w2_ref, out_ref, rw_ref, xb_ref):
    e = pl.program_id(0)
    ni = pl.program_id(1)

    @pl.when((e == 0) & (ni == 0))
    def _init():
        xf = x_ref[...]
        logits = jnp.dot(xf, wg_ref[...], preferred_element_type=jnp.float32)
        colid = jax.lax.broadcasted_iota(jnp.int32, logits.shape, 1)
        m1 = jnp.max(logits, axis=1, keepdims=True)
        idx1 = jnp.min(jnp.where(logits == m1, colid, E), axis=1, keepdims=True)
        sel1 = colid == idx1
        l2 = jnp.where(sel1, -jnp.inf, logits)
        m2 = jnp.max(l2, axis=1, keepdims=True)
        idx2 = jnp.min(jnp.where(l2 == m2, colid, E), axis=1, keepdims=True)
        sel2 = colid == idx2
        t = jnp.exp(m2 - m1)
        w_top = 1.0 / (1.0 + t)
        w_sec = t / (1.0 + t)
        rw_ref[...] = jnp.where(sel1, w_top, jnp.where(sel2, w_sec, 0.0))
        out_ref[...] = jnp.zeros_like(out_ref)
        xb_ref[...] = xf.astype(jnp.bfloat16)

    xb = xb_ref[...]
    a = jnp.dot(xb, w1_ref[0], preferred_element_type=jnp.float32,
                precision=jax.lax.Precision.DEFAULT)
    b = jnp.dot(xb, w3_ref[0], preferred_element_type=jnp.float32,
                precision=jax.lax.Precision.DEFAULT)
    h = (a * jax.nn.sigmoid(a)) * b
    colid = jax.lax.broadcasted_iota(jnp.int32, rw_ref.shape, 1)
    w_col = jnp.sum(jnp.where(colid == e, rw_ref[...], 0.0), axis=1, keepdims=True)
    out_ref[...] += jnp.dot(
        h * w_col, w2_ref[0],
        preferred_element_type=jnp.float32,
        precision=jax.lax.Precision.DEFAULT)


@jax.jit
def _moe(xf, Wg, W1, W3, W2):
    T = xf.shape[0]
    return pl.pallas_call(
        _moe_body,
        grid=(E, NI),
        in_specs=[
            pl.BlockSpec((T, H), lambda e, ni: (0, 0)),
            pl.BlockSpec((H, E), lambda e, ni: (0, 0)),
            pl.BlockSpec((1, H, TN), lambda e, ni: (e, 0, ni)),
            pl.BlockSpec((1, H, TN), lambda e, ni: (e, 0, ni)),
            pl.BlockSpec((1, TN, H), lambda e, ni: (e, ni, 0)),
        ],
        out_specs=pl.BlockSpec((T, H), lambda e, ni: (0, 0)),
        out_shape=jax.ShapeDtypeStruct((T, H), jnp.float32),
        scratch_shapes=[pltpu.VMEM((T, E), jnp.float32),
                        pltpu.VMEM((T, H), jnp.bfloat16)],
        compiler_params=pltpu.CompilerParams(
            dimension_semantics=("arbitrary", "arbitrary"),
        ),
    )(xf, Wg, W1, W3, W2)


def kernel(x, Wg, W1, W3, W2):
    B, S, Hd = x.shape
    xf = x.reshape(-1, Hd)
    out = _moe(xf, Wg, W1, W3, W2)
    return out.reshape(B, S, Hd)
